# baseline (speedup 1.0000x reference)
import jax
import jax.numpy as jnp
from jax import lax
from jax.experimental import pallas as pl

N = 10000
E = 160000
NP = 10240
KB = 1024
BN = 1024
BE = 640
MB = 1000


def _mm1(x, w, b):
    b2d = jnp.broadcast_to(b.reshape(1, -1), (8, b.shape[0]))

    def body(x_ref, w_ref, b_ref, o):
        o[...] = jnp.dot(x_ref[...], w_ref[...],
                         preferred_element_type=jnp.float32) + b_ref[0:1, :]

    return pl.pallas_call(
        body,
        grid=(N // MB,),
        in_specs=[pl.BlockSpec((MB, 256), lambda i: (i, 0)),
                  pl.BlockSpec((256, 128), lambda i: (0, 0)),
                  pl.BlockSpec((8, 128), lambda i: (0, 0))],
        out_specs=pl.BlockSpec((MB, 128), lambda i: (i, 0)),
        out_shape=jax.ShapeDtypeStruct((N, 128), jnp.float32),
    )(x, w, b2d)


def _gather_msgs(whp, srcm, d):
    def body(s_ref, w_ref, o):
        for r in range(8):
            ids = s_ref[0, r]
            acc = jnp.zeros((BE // 8, d), jnp.float32)
            for nb in range(NP // KB):
                rows = nb * KB + lax.broadcasted_iota(
                    jnp.int32, (BE // 8, KB), 1)
                oh = (ids[:, None] == rows).astype(jnp.bfloat16)
                acc += jnp.dot(oh, w_ref[pl.ds(nb * KB, KB), :],
                               preferred_element_type=jnp.float32)
            o[pl.ds(r * (BE // 8), BE // 8), :] = acc.astype(jnp.bfloat16)

    return pl.pallas_call(
        body,
        grid=(E // BE,),
        in_specs=[pl.BlockSpec((1, 8, BE // 8), lambda j: (j, 0, 0)),
                  pl.BlockSpec((NP, d), lambda j: (0, 0))],
        out_specs=pl.BlockSpec((BE, d), lambda j: (j, 0)),
        out_shape=jax.ShapeDtypeStruct((E, d), jnp.bfloat16),
    )(srcm, whp)


def _scatter_sum(g, dstm, d):
    def body(d_ref, g_ref, s_ref, c_ref):
        j = pl.program_id(1)

        @pl.when(j == 0)
        def _():
            s_ref[...] = jnp.zeros_like(s_ref)
            c_ref[...] = jnp.zeros_like(c_ref)

        i = pl.program_id(0)
        s_acc = jnp.zeros((BN, d), jnp.float32)
        c_acc = jnp.zeros((BN,), jnp.float32)
        for r in range(8):
            ids = d_ref[0, r]
            rows = i * BN + lax.broadcasted_iota(
                jnp.int32, (BN, BE // 8), 0)
            mask = rows == ids[None, :]
            oh = mask.astype(jnp.bfloat16)
            s_acc += jnp.dot(oh, g_ref[pl.ds(r * (BE // 8), BE // 8), :],
                             preferred_element_type=jnp.float32)
            c_acc += jnp.sum(mask.astype(jnp.float32), axis=1)
        s_ref[...] += s_acc
        c_ref[...] += jnp.broadcast_to(c_acc[:, None], (BN, 128))

    return pl.pallas_call(
        body,
        grid=(NP // BN, E // BE),
        in_specs=[pl.BlockSpec((1, 8, BE // 8), lambda i, j: (j, 0, 0)),
                  pl.BlockSpec((BE, d), lambda i, j: (j, 0))],
        out_specs=[pl.BlockSpec((BN, d), lambda i, j: (i, 0)),
                   pl.BlockSpec((BN, 128), lambda i, j: (i, 0))],
        out_shape=[jax.ShapeDtypeStruct((NP, d), jnp.float32),
                   jax.ShapeDtypeStruct((NP, 128), jnp.float32)],
    )(dstm, g)


def _mm2(s, c, w, b):
    b2d = jnp.broadcast_to(b.reshape(1, -1), (8, b.shape[0]))

    def body(s_ref, c_ref, w_ref, b_ref, o):
        x = s_ref[...] / jnp.maximum(c_ref[...], 1.0)
        x = jnp.where(x > 0, x, 0.01 * x)
        o[...] = jnp.dot(x, w_ref[...],
                         preferred_element_type=jnp.float32) + b_ref[0:1, :]

    return pl.pallas_call(
        body,
        grid=(N // MB,),
        in_specs=[pl.BlockSpec((MB, 128), lambda i: (i, 0)),
                  pl.BlockSpec((MB, 128), lambda i: (i, 0)),
                  pl.BlockSpec((128, 256), lambda i: (0, 0)),
                  pl.BlockSpec((8, 256), lambda i: (0, 0))],
        out_specs=pl.BlockSpec((MB, 256), lambda i: (i, 0)),
        out_shape=jax.ShapeDtypeStruct((N, 256), jnp.float32),
    )(s, c, w, b2d)


def _final_mean(s, c):
    def body(s_ref, c_ref, o):
        dd = jnp.maximum(c_ref[...], 1.0)
        o[...] = jnp.concatenate(
            [s_ref[:, :128] / dd, s_ref[:, 128:] / dd], axis=1)

    return pl.pallas_call(
        body,
        grid=(N // MB,),
        in_specs=[pl.BlockSpec((MB, 256), lambda i: (i, 0)),
                  pl.BlockSpec((MB, 128), lambda i: (i, 0))],
        out_specs=pl.BlockSpec((MB, 256), lambda i: (i, 0)),
        out_shape=jax.ShapeDtypeStruct((N, 256), jnp.float32),
    )(s, c)


def _pad_cast(x, d):
    return jnp.concatenate(
        [x.astype(jnp.bfloat16), jnp.zeros((NP - N, d), jnp.bfloat16)])


def kernel(edge_cd_src, edge_cd_dst, edge_dc_src, edge_dc_dst,
           embed_chemical, embed_disease,
           W1_cd, b1_cd, W1_dc, b1_dc,
           W2_cd, b2_cd, W2_dc, b2_dc):
    cd_src = edge_cd_src.astype(jnp.int32).reshape(E // BE, 8, BE // 8)
    cd_dst = edge_cd_dst.astype(jnp.int32).reshape(E // BE, 8, BE // 8)
    dc_src = edge_dc_src.astype(jnp.int32).reshape(E // BE, 8, BE // 8)
    dc_dst = edge_dc_dst.astype(jnp.int32).reshape(E // BE, 8, BE // 8)

    wh = _mm1(embed_chemical, W1_cd, b1_cd)
    g1 = _gather_msgs(_pad_cast(wh, 128), cd_src, 128)
    s1, c1 = _scatter_sum(g1, cd_dst, 128)
    wh2 = _mm2(s1[:N], c1[:N], W2_dc, b2_dc)
    g2 = _gather_msgs(_pad_cast(wh2, 256), dc_src, 256)
    s2, c2 = _scatter_sum(g2, dc_dst, 256)
    return _final_mean(s2[:N], c2[:N])
